# Initial kernel scaffold; baseline (speedup 1.0000x reference)
import jax
import jax.numpy as jnp
from jax.experimental import pallas as pl


def kernel(xyz, dense, hash_table):
    raise NotImplementedError("write your pallas kernel here")


# speedup vs baseline: 5.9621x; 5.9621x over previous
import functools

import numpy as np
import jax
import jax.numpy as jnp
from jax import lax
from jax.experimental import pallas as pl
from jax.experimental.pallas import tpu as pltpu
from jax.experimental.pallas import tpu_sc as plsc


def _nextprime_(n):
    def _isp(k):
        if k < 2:
            return False
        if k % 2 == 0:
            return k == 2
        i = 3
        while i * i <= k:
            if k % i == 0:
                return False
            i += 2
        return True
    k = n + 1
    while not _isp(k):
        k += 1
    return k


N_LEVELS = 16
F = 4
T = _nextprime_(2 ** 19)
DELTA = T - 2 ** 19
M19 = (1 << 19) - 1
P1, P2 = 19349663, 83492791
EN = [int(16 * 1.38 ** i) for i in range(N_LEVELS)]
CNT = [n ** 3 for n in EN]
CSUM = np.cumsum(CNT)
SH = next(i for i in range(N_LEVELS) if CNT[i] > T)
H = N_LEVELS - SH
DENSE_ROWS = int(CSUM[SH - 1])
ES = [np.float32(1.0 / (n - 1)) for n in EN]
RECIP = [np.float32(1.0 / np.float64(e)) for e in ES]
INV_T = np.float32(1.0) / np.float32(T)

N_PTS = 65536
NW = 32
C = 1024
NCHUNK = N_PTS // (NW * C)
G = C // 16
NIDX = 8 * C
NROW = NIDX // 128
KDEPTH = 8

I = np.int32
FL = np.float32
_i32 = jnp.int32
_f32 = jnp.float32


def _hash_split(c, P):
    ph, plo = P >> 19, P & M19
    t = c * I(plo)
    hi = c * I(ph) + (t >> I(19))
    lo = t & I(M19)
    return hi, lo


def _loop(n, fn):
    def b(_, j):
        fn(j)
        return j + I(1)
    lax.fori_loop(I(0), I(n), b, I(0))


def _sc_body(xyzT, dsum, hsum, out, xn_v, idx_v, gbuf, ofx_v, ofy_v,
             col_v, sem, sem2):


    def _copy(src, dst):
        c = pltpu.make_async_copy(src, dst, sem2)
        c.start()
        c.wait()
    wid = lax.axis_index("s") * I(2) + lax.axis_index("c")

    def chunk_body(chunk):
        base = wid * I(C * NCHUNK) + chunk * I(C)
        _copy(xyzT.at[:, pl.ds(base, C)], xn_v)

        def norm_body(g):
            s = pl.ds(g * I(16), 16)
            for d in range(3):
                xn_v[d, s] = xn_v[d, s] * FL(0.25) + FL(0.5)
        _loop(G, norm_body)
        _copy(xn_v, out.at[pl.ds(I(0), 3), pl.ds(base, C)])

        for l in range(N_LEVELS):
            n = EN[l]
            recip = RECIP[l]
            tbl = dsum if l < SH else hsum

            def idx_body(g, l=l, n=n, recip=recip):
                s = pl.ds(g * I(16), 16)
                cs = (g & I(7)) * I(16)
                r0 = g >> I(3)
                cc = []
                for d in range(3):
                    flt = xn_v[d, s] * recip
                    ic = flt.astype(_i32)
                    c0 = jnp.minimum(ic, I(n - 1))
                    c1 = jnp.minimum(ic + I(1), I(n - 1))
                    off = flt - c0.astype(_f32)
                    if d == 0:
                        ofx_v[s] = off
                    elif d == 1:
                        ofy_v[s] = off
                    cc.append((c0, c1))
                if l < SH:
                    bl = 0 if l == 0 else int(CSUM[l - 1])
                    ax = [cc[0][0] * I(n * n) + I(bl),
                          cc[0][1] * I(n * n) + I(bl)]
                    by = [cc[1][0] * I(n), cc[1][1] * I(n)]
                    for c in range(8):
                        ox, oy, oz = c >> 2, (c >> 1) & 1, c & 1
                        idx = ax[ox] + by[oy] + cc[2][oz]
                        idx_v[I(c * 8) + r0, pl.ds(cs, 16)] = idx
                else:
                    hy = [_hash_split(cc[1][i], P1) for i in range(2)]
                    hz = [_hash_split(cc[2][i], P2) for i in range(2)]
                    lyz = [[hy[a][1] ^ hz[b][1] for b in range(2)]
                           for a in range(2)]
                    h21 = [[(hy[a][0] ^ hz[b][0]) * I(DELTA)
                            for b in range(2)] for a in range(2)]
                    for c in range(8):
                        ox, oy, oz = c >> 2, (c >> 1) & 1, c & 1
                        lo = lyz[oy][oz] ^ cc[0][ox]
                        r = lo - h21[oy][oz] + I(DELTA * T)
                        q = (r.astype(_f32) * INV_T).astype(_i32)
                        m = r - q * I(T)
                        m = jnp.where(m < I(0), m + I(T), m)
                        m = jnp.where(m >= I(T), m - I(T), m)
                        idx_v[I(c * 8) + r0, pl.ds(cs, 16)] = m + I((l - SH) * T)
            _loop(G, idx_body)

            def cp(j, tbl=tbl):
                return pltpu.make_async_copy(
                    tbl.at[idx_v.at[j]],
                    gbuf.at[pl.ds(j * I(128), 128)],
                    sem)

            def fire_body(j):
                cp(j).start()

                @pl.when(j >= I(KDEPTH))
                def _():
                    cp(j - I(KDEPTH)).wait()
            _loop(NROW, fire_body)

            def tail_body(t):
                cp(I(NROW - KDEPTH) + t).wait()
            _loop(KDEPTH, tail_body)

            def comb_body(g):
                s = pl.ds(g * I(16), 16)
                ox1 = ofx_v[s]
                oy1 = ofy_v[s]
                ox0 = FL(1.0) - ox1
                oy0 = FL(1.0) - oy1
                w = [ox0 * oy0, ox0 * oy1, ox1 * oy0, ox1 * oy1]
                rb = g * I(16)
                vacc = jnp.zeros((16,), _f32)
                for c in range(8):
                    sv = gbuf[pl.ds(I(c * C) + rb, 16)]
                    vacc = vacc + w[c >> 1] * sv
                col_v[I(0), s] = vacc
            _loop(G, comb_body)

            _copy(col_v, out.at[pl.ds(I(3 + l), 1), pl.ds(base, C)])

    _loop(NCHUNK, chunk_body)


_RS_M = np.zeros((128, 32), np.float32)
_RS_M[np.arange(128), np.arange(128) // 4] = 1.0


def _rowsum_body(x_ref, m_ref, o_ref):
    o_ref[...] = jnp.dot(x_ref[...], m_ref[...],
                         preferred_element_type=jnp.float32)


def _rowsum(x2d, bm=2048):
    rb = x2d.shape[0]
    grid = (rb + bm - 1) // bm
    return pl.pallas_call(
        _rowsum_body,
        grid=(grid,),
        in_specs=[pl.BlockSpec((bm, 128), lambda i: (i, 0)),
                  pl.BlockSpec((128, 32), lambda i: (0, 0))],
        out_specs=pl.BlockSpec((bm, 32), lambda i: (i, 0)),
        out_shape=jax.ShapeDtypeStruct((rb, 32), jnp.float32),
    )(x2d, jnp.asarray(_RS_M))


@jax.jit
def _hash_encode(xyzT, dsum, hsum):
    mesh = plsc.VectorSubcoreMesh(core_axis_name="c", subcore_axis_name="s")
    fn = functools.partial(
        pl.kernel,
        mesh=mesh,
        out_type=jax.ShapeDtypeStruct((3 + N_LEVELS, N_PTS), jnp.float32),
        scratch_types=[
            pltpu.VMEM((3, C), jnp.float32),
            pltpu.VMEM((NROW, 128), jnp.int32),
            pltpu.VMEM((NIDX,), jnp.float32),
            pltpu.VMEM((C,), jnp.float32),
            pltpu.VMEM((C,), jnp.float32),
            pltpu.VMEM((1, C), jnp.float32),
            pltpu.SemaphoreType.DMA,
            pltpu.SemaphoreType.DMA,
        ],
    )(_sc_body)
    return fn(xyzT, dsum, hsum)


def kernel(xyz, dense, hash_table):
    with jax.enable_x64(False):
        xyzT = jnp.asarray(xyz, jnp.float32).reshape(-1, 3).T
        dsum = _rowsum(dense.reshape(-1, 128)).reshape(-1)
        hflat = hash_table.reshape(-1)
        pad = (-hflat.shape[0]) % 128
        hsum = _rowsum(jnp.pad(hflat, (0, pad)).reshape(-1, 128)).reshape(-1)
        outT = _hash_encode(xyzT, dsum, hsum)
        return outT.T
